# baseline (speedup 1.0000x reference)
import functools

import jax
import jax.numpy as jnp
from jax import lax
from jax.experimental import pallas as pl
from jax.experimental.pallas import tpu as pltpu
from jax.experimental.pallas import tpu_sc as plsc
from jax.experimental.layout import Layout as _Layout
from jax.experimental.layout import with_layout_constraint as _with_layout

_V = 1000000
_D = 64
_L = 200
_B = 4096
_DP = 16
_H = 104
_LP = 2 * _H

_PACK = 128 // _DP
_VW = _V // _PACK
_PROJ_BLK = 1000


def _proj_body(e_ref, g_ref, out_ref):
    ew = e_ref[...].reshape(_PROJ_BLK, _PACK * _D)
    out_ref[...] = jnp.dot(ew, g_ref[...], preferred_element_type=jnp.float32)


def _project(emb, wp):
    e3 = emb.reshape(_VW, _PACK, _D)
    g = jnp.kron(jnp.eye(_PACK, dtype=jnp.float32), wp)
    return pl.pallas_call(
        _proj_body,
        grid=(_VW // _PROJ_BLK,),
        in_specs=[
            pl.BlockSpec((_PROJ_BLK, _PACK, _D), lambda i: (i, 0, 0)),
            pl.BlockSpec((_PACK * _D, _PACK * _DP), lambda i: (0, 0)),
        ],
        out_specs=pl.BlockSpec((_PROJ_BLK, _PACK * _DP), lambda i: (i, 0)),
        out_shape=jax.ShapeDtypeStruct((_VW, _PACK * _DP), jnp.float32),
    )(e3, g)


@functools.cache
def _make_sc_pool():
    info = plsc.get_sparse_core_info()
    nc, ns = info.num_cores, info.num_subcores
    nw = nc * ns
    bpw = _B // nw
    mesh = plsc.VectorSubcoreMesh(core_axis_name="c", subcore_axis_name="s")

    @functools.partial(
        pl.kernel, mesh=mesh,
        out_type=jax.ShapeDtypeStruct((_B, _DP), jnp.float32),
        compiler_params=pltpu.CompilerParams(use_tc_tiling_on_sc=False),
        scratch_types=[
            pltpu.VMEM((bpw, 2, _H), jnp.int32),
            pltpu.VMEM((2, _LP, _DP), jnp.float32),
            pltpu.VMEM((bpw, _DP), jnp.float32),
            pltpu.VMEM((_DP,), jnp.float32),
            pltpu.SemaphoreType.DMA,
            pltpu.SemaphoreType.DMA,
        ],
    )
    def pool(p_hbm, xp_hbm, bias_hbm, out_hbm,
             idx_v, rows_v, out_v, bias_v, sem0, sem1):
        wid = lax.axis_index("s") * nc + lax.axis_index("c")
        base = wid * bpw
        pltpu.sync_copy(xp_hbm.at[pl.ds(base, bpw)], idx_v)
        pltpu.sync_copy(bias_hbm, bias_v)
        bias = bias_v[...]
        sems = (sem0, sem1)

        def copies(r, buf):
            sem = sems[buf]
            c0 = pltpu.make_async_copy(
                p_hbm.at[idx_v.at[r, 0]], rows_v.at[buf, pl.ds(0, _H)], sem)
            c1 = pltpu.make_async_copy(
                p_hbm.at[idx_v.at[r, 1]], rows_v.at[buf, pl.ds(_H, _H)], sem)
            return c0, c1

        def start(r, buf):
            c0, c1 = copies(r, buf)
            c0.start()
            c1.start()

        def finish(r, buf):
            c0, c1 = copies(r, buf)
            c0.wait()
            c1.wait()

        zero = jnp.zeros((_DP,), jnp.float32)

        def accum(buf):
            def acc_body(jj, accs):
                j = jj * 8
                a0, a1, a2, a3 = accs
                a0 = a0 + rows_v[buf, j]
                a1 = a1 + rows_v[buf, j + 1]
                a2 = a2 + rows_v[buf, j + 2]
                a3 = a3 + rows_v[buf, j + 3]
                a0 = a0 + rows_v[buf, j + 4]
                a1 = a1 + rows_v[buf, j + 5]
                a2 = a2 + rows_v[buf, j + 6]
                a3 = a3 + rows_v[buf, j + 7]
                return (a0, a1, a2, a3)

            a0, a1, a2, a3 = lax.fori_loop(
                0, _L // 8, acc_body, (bias, zero, zero, zero))
            return (a0 + a1) + (a2 + a3)

        start(0, 0)
        start(1, 1)

        def pair_body(i, carry):
            r0 = 2 * i
            r1 = r0 + 1
            finish(r0, 0)
            acc0 = accum(0)
            out_v[r0] = acc0
            start(jnp.minimum(r0 + 2, bpw - 1), 0)
            finish(r1, 1)
            acc1 = accum(1)
            out_v[r1] = acc1
            start(jnp.minimum(r1 + 2, bpw - 1), 1)
            return carry

        lax.fori_loop(0, bpw // 2, pair_body, 0)
        finish(bpw - 1, 0)
        finish(bpw - 1, 1)
        pltpu.sync_copy(out_v, out_hbm.at[pl.ds(base, bpw)])

    return pool


@functools.cache
def _make_sc_repack():
    info = plsc.get_sparse_core_info()
    nc, ns = info.num_cores, info.num_subcores
    nw = nc * ns
    mesh = plsc.VectorSubcoreMesh(core_axis_name="c", subcore_axis_name="s")
    C = 256
    NCH = (_VW + C - 1) // C
    TPW = (NCH + nw - 1) // nw
    LAST = _VW - C

    @functools.partial(
        pl.kernel, mesh=mesh,
        out_type=jax.ShapeDtypeStruct((_V, _DP), jnp.float32),
        compiler_params=pltpu.CompilerParams(use_tc_tiling_on_sc=False),
        scratch_types=[
            pltpu.VMEM((C, _PACK * _DP), jnp.float32),
            pltpu.VMEM((C * _PACK, _DP), jnp.float32),
        ],
    )
    def repack(p_hbm, out_hbm, inb, outb):
        wid = lax.axis_index("s") * nc + lax.axis_index("c")

        def chunk_body(t, carry):
            st = jnp.minimum((wid + nw * t) * C, LAST)
            pltpu.sync_copy(p_hbm.at[pl.ds(st, C)], inb)

            def row_body(j, c2):
                for a in range(_PACK):
                    outb[_PACK * j + a] = inb[j, pl.ds(_DP * a, _DP)]
                return c2

            lax.fori_loop(0, C, row_body, 0)
            pltpu.sync_copy(outb, out_hbm.at[pl.ds(_PACK * st, _PACK * C)])
            return carry

        lax.fori_loop(0, TPW, chunk_body, 0)

    return repack


def kernel(x, emb_table, W, b):
    wp = jnp.zeros((_D, _DP), jnp.float32).at[:, :2].set(W.T * (1.0 / _L))
    p128 = _project(emb_table, wp)
    p = _make_sc_repack()(p128)
    xi = x.astype(jnp.int32)
    xp = jnp.pad(xi, ((0, 0), (0, _LP - _L))).reshape(_B, 2, _H)
    bias_pad = jnp.zeros((_DP,), jnp.float32).at[:2].set(b)
    out_pad = _make_sc_pool()(p, xp, bias_pad)
    return out_pad[:, :2]


# speedup vs baseline: 1.1212x; 1.1212x over previous
import functools

import jax
import jax.numpy as jnp
from jax import lax
from jax.experimental import pallas as pl
from jax.experimental.pallas import tpu as pltpu
from jax.experimental.pallas import tpu_sc as plsc
from jax.experimental.layout import Layout as _Layout
from jax.experimental.layout import with_layout_constraint as _with_layout

_V = 1000000
_D = 64
_L = 200
_B = 4096
_DP = 16
_H = 104
_LP = 2 * _H

_PACK = 128 // _DP
_VW = _V // _PACK
_PROJ_BLK = 1000


def _proj_body(e_ref, g_ref, out_ref):
    ew = e_ref[...].reshape(_PROJ_BLK, _PACK * _D)
    out_ref[...] = jnp.dot(ew, g_ref[...], preferred_element_type=jnp.float32)


def _project(emb, wp):
    e3 = emb.reshape(_VW, _PACK, _D)
    g = jnp.kron(jnp.eye(_PACK, dtype=jnp.float32), wp)
    return pl.pallas_call(
        _proj_body,
        grid=(_VW // _PROJ_BLK,),
        in_specs=[
            pl.BlockSpec((_PROJ_BLK, _PACK, _D), lambda i: (i, 0, 0)),
            pl.BlockSpec((_PACK * _D, _PACK * _DP), lambda i: (0, 0)),
        ],
        out_specs=pl.BlockSpec((_PROJ_BLK, _PACK * _DP), lambda i: (i, 0)),
        out_shape=jax.ShapeDtypeStruct((_VW, _PACK * _DP), jnp.float32),
    )(e3, g)


@functools.cache
def _make_sc_pool():
    info = plsc.get_sparse_core_info()
    nc, ns = info.num_cores, info.num_subcores
    nw = nc * ns
    bpw = _B // nw
    mesh = plsc.VectorSubcoreMesh(core_axis_name="c", subcore_axis_name="s")

    @functools.partial(
        pl.kernel, mesh=mesh,
        out_type=jax.ShapeDtypeStruct((_B, _DP), jnp.float32),
        compiler_params=pltpu.CompilerParams(use_tc_tiling_on_sc=False),
        scratch_types=[
            pltpu.VMEM((bpw, 2, _H), jnp.int32),
            pltpu.VMEM((2, _LP, _DP), jnp.float32),
            pltpu.VMEM((bpw, _DP), jnp.float32),
            pltpu.VMEM((_DP,), jnp.float32),
            pltpu.SemaphoreType.DMA,
            pltpu.SemaphoreType.DMA,
        ],
    )
    def pool(p_hbm, xp_hbm, bias_hbm, out_hbm,
             idx_v, rows_v, out_v, bias_v, sem0, sem1):
        wid = lax.axis_index("s") * nc + lax.axis_index("c")
        base = wid * bpw
        pltpu.sync_copy(xp_hbm.at[pl.ds(base, bpw)], idx_v)
        pltpu.sync_copy(bias_hbm, bias_v)
        bias = bias_v[...]
        sems = (sem0, sem1)

        def copies(r, buf):
            sem = sems[buf]
            c0 = pltpu.make_async_copy(
                p_hbm.at[idx_v.at[r, 0]], rows_v.at[buf, pl.ds(0, _H)], sem)
            c1 = pltpu.make_async_copy(
                p_hbm.at[idx_v.at[r, 1]], rows_v.at[buf, pl.ds(_H, _H)], sem)
            return c0, c1

        def start(r, buf):
            c0, c1 = copies(r, buf)
            c0.start()
            c1.start()

        def finish(r, buf):
            c0, c1 = copies(r, buf)
            c0.wait()
            c1.wait()

        zero = jnp.zeros((_DP,), jnp.float32)

        def accum(buf):
            def acc_body(jj, accs):
                j = jj * 8
                a0, a1, a2, a3 = accs
                a0 = a0 + rows_v[buf, j]
                a1 = a1 + rows_v[buf, j + 1]
                a2 = a2 + rows_v[buf, j + 2]
                a3 = a3 + rows_v[buf, j + 3]
                a0 = a0 + rows_v[buf, j + 4]
                a1 = a1 + rows_v[buf, j + 5]
                a2 = a2 + rows_v[buf, j + 6]
                a3 = a3 + rows_v[buf, j + 7]
                return (a0, a1, a2, a3)

            a0, a1, a2, a3 = lax.fori_loop(
                0, _L // 8, acc_body, (bias, zero, zero, zero))
            return (a0 + a1) + (a2 + a3)

        start(0, 0)
        start(1, 1)

        def pair_body(i, carry):
            r0 = 2 * i
            r1 = r0 + 1
            finish(r0, 0)
            acc0 = accum(0)
            out_v[r0] = acc0
            start(jnp.minimum(r0 + 2, bpw - 1), 0)
            finish(r1, 1)
            acc1 = accum(1)
            out_v[r1] = acc1
            start(jnp.minimum(r1 + 2, bpw - 1), 1)
            return carry

        lax.fori_loop(0, bpw // 2, pair_body, 0)
        finish(bpw - 1, 0)
        finish(bpw - 1, 1)
        pltpu.sync_copy(out_v, out_hbm.at[pl.ds(base, bpw)])

    return pool


def kernel(x, emb_table, W, b):
    wp = jnp.zeros((_D, _DP), jnp.float32).at[:, :2].set(W.T * (1.0 / _L))
    p128 = _project(emb_table, wp)
    p = _with_layout(p128.reshape(_V, _DP), _Layout((0, 1)))
    xi = x.astype(jnp.int32)
    xp = jnp.pad(xi, ((0, 0), (0, _LP - _L))).reshape(_B, 2, _H)
    bias_pad = jnp.zeros((_DP,), jnp.float32).at[:2].set(b)
    out_pad = _make_sc_pool()(p, xp, bias_pad)
    return out_pad[:, :2]


# speedup vs baseline: 1.1385x; 1.0154x over previous
import functools

import jax
import jax.numpy as jnp
from jax import lax
from jax.experimental import pallas as pl
from jax.experimental.pallas import tpu as pltpu
from jax.experimental.pallas import tpu_sc as plsc
from jax.experimental.layout import Layout as _Layout
from jax.experimental.layout import with_layout_constraint as _with_layout

_V = 1000000
_D = 64
_L = 200
_B = 4096
_DP = 8
_H = 104
_LP = 2 * _H

_PACK = 128 // _DP
_VW = _V // _PACK
_PROJ_BLK = 512


def _proj_body(e_ref, g_ref, out_ref):
    ew = e_ref[...].reshape(_PROJ_BLK, _PACK * _D)
    out_ref[...] = jnp.dot(ew, g_ref[...], preferred_element_type=jnp.float32)


def _project(emb, wp):
    e3 = emb.reshape(_VW, _PACK, _D)
    g = jnp.kron(jnp.eye(_PACK, dtype=jnp.float32), wp)
    return pl.pallas_call(
        _proj_body,
        grid=((_VW + _PROJ_BLK - 1) // _PROJ_BLK,),
        in_specs=[
            pl.BlockSpec((_PROJ_BLK, _PACK, _D), lambda i: (i, 0, 0)),
            pl.BlockSpec((_PACK * _D, 128), lambda i: (0, 0)),
        ],
        out_specs=pl.BlockSpec((_PROJ_BLK, 128), lambda i: (i, 0)),
        out_shape=jax.ShapeDtypeStruct((_VW, 128), jnp.float32),
    )(e3, g)


@functools.cache
def _make_sc_pool():
    info = plsc.get_sparse_core_info()
    nc, ns = info.num_cores, info.num_subcores
    nw = nc * ns
    bpw = _B // nw
    mesh = plsc.VectorSubcoreMesh(core_axis_name="c", subcore_axis_name="s")

    @functools.partial(
        pl.kernel, mesh=mesh,
        out_type=jax.ShapeDtypeStruct((_B, _DP), jnp.float32),
        compiler_params=pltpu.CompilerParams(use_tc_tiling_on_sc=False),
        scratch_types=[
            pltpu.VMEM((bpw, 2, _H), jnp.int32),
            pltpu.VMEM((bpw, 2, _H), jnp.int32),
            pltpu.VMEM((2, _LP, _DP), jnp.float32),
            pltpu.VMEM_SHARED((ns * bpw + 8, _DP), jnp.float32),
            pltpu.SemaphoreType.DMA,
            pltpu.SemaphoreType.DMA,
            pltpu.SemaphoreType.DMA,
            pltpu.SemaphoreType.DMA,
        ],
    )
    def pool(p_hbm, xp_hbm, slots_hbm, zero_hbm, out_hbm,
             idx_v, slot_v, rows_v, acc_sh,
             gsem0, gsem1, ssem0, ssem1):
        cid = lax.axis_index("c")
        sid = lax.axis_index("s")
        wid = sid * nc + cid
        base = wid * bpw
        pltpu.sync_copy(xp_hbm.at[pl.ds(base, bpw)], idx_v)
        pltpu.sync_copy(slots_hbm.at[pl.ds(base, bpw)], slot_v)
        pltpu.sync_copy(zero_hbm, acc_sh.at[pl.ds(sid * bpw, bpw)])

        gsems = (gsem0, gsem1)
        ssems = (ssem0, ssem1)

        def gathers(r, buf):
            sem = gsems[buf]
            c0 = pltpu.make_async_copy(
                p_hbm.at[idx_v.at[r, 0]], rows_v.at[buf, pl.ds(0, _H)], sem)
            c1 = pltpu.make_async_copy(
                p_hbm.at[idx_v.at[r, 1]], rows_v.at[buf, pl.ds(_H, _H)], sem)
            return c0, c1

        def scatters(r, buf):
            sem = ssems[buf]
            c0 = pltpu.make_async_copy(
                rows_v.at[buf, pl.ds(0, _H)], acc_sh.at[slot_v.at[r, 0]], sem)
            c1 = pltpu.make_async_copy(
                rows_v.at[buf, pl.ds(_H, _H)], acc_sh.at[slot_v.at[r, 1]], sem)
            return c0, c1

        def start(cs, add=False):
            for c in cs:
                c.start(add=add)

        def wait(cs):
            for c in cs:
                c.wait()

        start(gathers(0, 0))
        start(gathers(1, 1))

        def pair_body(i, carry):
            r0 = 2 * i
            r1 = r0 + 1
            wait(gathers(r0, 0))
            start(scatters(r0, 0), add=True)
            wait(scatters(r0, 0))
            start(gathers(jnp.minimum(r0 + 2, bpw - 1), 0))
            wait(gathers(r1, 1))
            start(scatters(r1, 1), add=True)
            wait(scatters(r1, 1))
            start(gathers(jnp.minimum(r1 + 2, bpw - 1), 1))
            return carry

        lax.fori_loop(0, bpw // 2, pair_body, 0)
        wait(gathers(bpw - 1, 0))
        wait(gathers(bpw - 1, 1))
        pltpu.sync_copy(acc_sh.at[pl.ds(sid * bpw, bpw)],
                        out_hbm.at[pl.ds(base, bpw)])

    return pool


def kernel(x, emb_table, W, b):
    wp = jnp.zeros((_D, _DP), jnp.float32).at[:, :2].set(W.T * (1.0 / _L))
    p128 = _project(emb_table, wp)
    p = _with_layout(p128.reshape(_V, _DP), _Layout((0, 1)))
    xi = x.astype(jnp.int32)
    xp = jnp.pad(xi, ((0, 0), (0, _LP - _L))).reshape(_B, 2, _H)
    binfo = jnp.arange(_B, dtype=jnp.int32)
    slot = (binfo // (2 * 128)) * 128 + binfo % 128
    slots = jnp.broadcast_to(slot[:, None], (_B, _LP))
    lpos = jnp.arange(_LP, dtype=jnp.int32)[None, :]
    slots = jnp.where(lpos < _L, slots, 16 * 128).reshape(_B, 2, _H)
    zero = jnp.zeros((_B // 32, _DP), jnp.float32)
    out_pad = _make_sc_pool()(p, xp, slots, zero)
    return out_pad[:, :2] + b
